# baseline (speedup 1.0000x reference)
import functools

import jax
import jax.numpy as jnp
from jax import lax
from jax.experimental import pallas as pl
from jax.experimental.pallas import tpu as pltpu
from jax.experimental.pallas import tpu_sc as plsc

VOCAB = 1000000
D = 32
BATCH = 4096
SEQ = 200

NC = 2
NS = 16
NW = NC * NS
BW = BATCH // NW
NB = 1
NCHUNK = BW // NB
HALVES = ((0, 128), (128, 72))
NBUF = 4


TB = 512


def _tr_body(x_ref, o_ref):
    xt = x_ref[...].T
    o_ref[...] = jnp.concatenate(
        [xt, jnp.zeros((TB, 128 - D), jnp.float32)], axis=1
    )


def _make_detile():
    grid = (VOCAB + TB - 1) // TB
    return pl.pallas_call(
        _tr_body,
        grid=(grid,),
        in_specs=[pl.BlockSpec((D, TB), lambda i: (0, i))],
        out_specs=pl.BlockSpec((TB, 128), lambda i: (i, 0)),
        out_shape=jax.ShapeDtypeStruct((VOCAB, 128), jnp.float32),
    )


def _make_gather():
    mesh = plsc.VectorSubcoreMesh(core_axis_name="c", subcore_axis_name="s")

    @functools.partial(
        pl.kernel,
        mesh=mesh,
        compiler_params=pltpu.CompilerParams(use_tc_tiling_on_sc=False),
        out_type=jax.ShapeDtypeStruct((BATCH * SEQ, 4 * D), jnp.float32),
        scratch_types=[
            pltpu.VMEM((BW, SEQ), jnp.int32),
            pltpu.VMEM((NBUF, NB * SEQ, 128), jnp.float32),
            [pltpu.SemaphoreType.DMA] * NBUF,
            [pltpu.SemaphoreType.DMA] * NBUF,
        ],
    )
    def gather(idx_hbm, table_hbm, out_hbm, idx_v, stage_v, gsems, osems):
        wid = lax.axis_index("s") * NC + lax.axis_index("c")
        pltpu.sync_copy(idx_hbm.at[wid], idx_v)

        def gather_copies(c, slot):
            out = []
            for i in range(NB):
                for off, width in HALVES:
                    out.append(
                        pltpu.make_async_copy(
                            table_hbm.at[
                                idx_v.at[c * NB + i, pl.ds(off, width)]
                            ],
                            stage_v.at[slot, pl.ds(i * SEQ + off, width)],
                            gsems[slot],
                        )
                    )
            return out

        def out_copy(c, slot):
            base = (wid * BW + c * NB) * SEQ
            return pltpu.make_async_copy(
                stage_v.at[slot, pl.ds(0, NB * SEQ), pl.ds(0, D)],
                out_hbm.at[pl.ds(base, NB * SEQ), pl.ds(0, D)],
                osems[slot],
            )

        for c0 in range(2):
            for g in gather_copies(c0, c0):
                g.start()

        def outer(o, carry):
            for b in range(NBUF):
                c = NBUF * o + b
                for g in gather_copies(c, b):
                    g.wait()
                out_copy(c, b).start()
                nxt = (b + 2) % NBUF
                @pl.when(c >= 2)
                def _():
                    out_copy(c - 2, nxt).wait()
                @pl.when(c + 2 < NCHUNK)
                def _():
                    for g in gather_copies(c + 2, nxt):
                        g.start()
            return carry

        lax.fori_loop(0, NCHUNK // NBUF, outer, 0)
        for c0 in range(NCHUNK - 2, NCHUNK):
            out_copy(c0, c0 % NBUF).wait()

    return gather


_gather = _make_gather()
_detile = _make_detile()


def kernel(sentence, table):
    idx = sentence.astype(jnp.int32).reshape(NW, BW, SEQ)
    table_rm = _detile(jnp.swapaxes(table, 0, 1))
    out = _gather(idx, table_rm)
    return out[:, :D].reshape(BATCH, SEQ, D)


# speedup vs baseline: 2.1239x; 2.1239x over previous
import functools

import jax
import jax.numpy as jnp
from jax import lax
from jax.experimental import pallas as pl
from jax.experimental.pallas import tpu as pltpu
from jax.experimental.pallas import tpu_sc as plsc

VOCAB = 1000000
D = 32
BATCH = 4096
SEQ = 200

NC = 2
NS = 16
NW = NC * NS
BW = BATCH // NW
NB = 4
NCHUNK = BW // NB
HALVES = ((0, 128), (128, 72))
NBUF = 4


def _make_gather():
    mesh = plsc.VectorSubcoreMesh(core_axis_name="c", subcore_axis_name="s")

    @functools.partial(
        pl.kernel,
        mesh=mesh,
        compiler_params=pltpu.CompilerParams(use_tc_tiling_on_sc=False),
        out_type=jax.ShapeDtypeStruct((BATCH * SEQ, 4 * D), jnp.float32),
        scratch_types=[
            pltpu.VMEM((BW, SEQ), jnp.int32),
            pltpu.VMEM((NBUF, NB * SEQ, D), jnp.float32),
            [pltpu.SemaphoreType.DMA] * NBUF,
            [pltpu.SemaphoreType.DMA] * NBUF,
        ],
    )
    def gather(idx_hbm, table_hbm, out_hbm, idx_v, stage_v, gsems, osems):
        wid = lax.axis_index("s") * NC + lax.axis_index("c")
        pltpu.sync_copy(idx_hbm.at[wid], idx_v)

        def gather_copies(c, slot):
            out = []
            for i in range(NB):
                for off, width in HALVES:
                    out.append(
                        pltpu.make_async_copy(
                            table_hbm.at[
                                idx_v.at[c * NB + i, pl.ds(off, width)]
                            ],
                            stage_v.at[slot, pl.ds(i * SEQ + off, width)],
                            gsems[slot],
                        )
                    )
            return out

        def out_copy(c, slot):
            base = (wid * BW + c * NB) * SEQ
            return pltpu.make_async_copy(
                stage_v.at[slot],
                out_hbm.at[pl.ds(base, NB * SEQ), pl.ds(0, D)],
                osems[slot],
            )

        for c0 in range(2):
            for g in gather_copies(c0, c0):
                g.start()

        def outer(o, carry):
            for b in range(NBUF):
                c = NBUF * o + b
                for g in gather_copies(c, b):
                    g.wait()
                out_copy(c, b).start()
                nxt = (b + 2) % NBUF
                @pl.when(c >= 2)
                def _():
                    out_copy(c - 2, nxt).wait()
                @pl.when(c + 2 < NCHUNK)
                def _():
                    for g in gather_copies(c + 2, nxt):
                        g.start()
            return carry

        lax.fori_loop(0, NCHUNK // NBUF, outer, 0)
        for c0 in range(NCHUNK - 2, NCHUNK):
            out_copy(c0, c0 % NBUF).wait()

    return gather


_gather = _make_gather()


def kernel(sentence, table):
    idx = sentence.astype(jnp.int32).reshape(NW, BW, SEQ)
    out = _gather(idx, table)
    return out[:, :D].reshape(BATCH, SEQ, D)
